# baseline (speedup 1.0000x reference)
import functools

import jax
import jax.numpy as jnp
from jax import lax
from jax.experimental import pallas as pl
from jax.experimental.pallas import tpu as pltpu
from jax.experimental.pallas import tpu_sc as plsc

_G = 16
_LANE = 128
_CHUNK = 128


def _sc_segment_sum(n, f, n_chunks, chunks_per_tile, rem_base, rem):
    mesh = plsc.VectorSubcoreMesh(core_axis_name="c", subcore_axis_name="s")
    nc = mesh.num_cores

    @functools.partial(
        pl.kernel,
        out_type=jax.ShapeDtypeStruct((nc, _G, f), jnp.float32),
        mesh=mesh,
        scratch_types=[
            pltpu.VMEM((_CHUNK, f), jnp.float32),
            pltpu.VMEM((_CHUNK,), jnp.int32),
            pltpu.VMEM((rem, f), jnp.float32),
            pltpu.VMEM((rem,), jnp.int32),
            pltpu.VMEM_SHARED((_G, f), jnp.float32),
        ],
    )
    def seg_sum(x_hbm, seg_hbm, zeros_hbm, out_hbm,
                chunk_v, idx_v, rem_v, ridx_v, acc_sh):
        c_id = lax.axis_index("c")
        s_id = lax.axis_index("s")
        wid = s_id * nc + c_id
        n_workers = nc * mesh.num_subcores

        @pl.when(s_id == 0)
        def _init():
            pltpu.sync_copy(zeros_hbm, acc_sh)

        plsc.subcore_barrier()

        for k in range(chunks_per_tile):
            cid = wid + k * n_workers

            @pl.when(cid < n_chunks)
            def _do():
                base = cid * _CHUNK
                pltpu.sync_copy(x_hbm.at[pl.ds(base, _CHUNK)], chunk_v)
                pltpu.sync_copy(seg_hbm.at[pl.ds(base, _CHUNK)], idx_v)
                pltpu.sync_copy(chunk_v, acc_sh.at[idx_v], add=True)

        @pl.when(wid == 0)
        def _remainder():
            pltpu.sync_copy(x_hbm.at[pl.ds(rem_base, rem)], rem_v)
            pltpu.sync_copy(seg_hbm.at[pl.ds(rem_base, rem)], ridx_v)
            pltpu.sync_copy(rem_v, acc_sh.at[ridx_v], add=True)

        plsc.subcore_barrier()

        @pl.when(s_id == 0)
        def _flush():
            pltpu.sync_copy(acc_sh, out_hbm.at[c_id])

    return seg_sum


def _head_kernel(part_ref, fc1w_ref, fc1b_ref, outw_ref, outb_ref, o_ref):
    pooled = part_ref[0] + part_ref[1]
    h = jnp.dot(pooled, fc1w_ref[...],
                preferred_element_type=jnp.float32) + fc1b_ref[0, :]
    logits = jnp.dot(h, outw_ref[...],
                     preferred_element_type=jnp.float32) + outb_ref[0, :]
    o_ref[...] = jax.nn.sigmoid(logits)


def kernel(x, edge_index, seg, kernel0, a_self0, a_neigh0, bias0,
           kernel1, a_self1, a_neigh1, bias1, fc1_W, fc1_b, out_W, out_b):
    n, f = x.shape
    pre = fc1_W.shape[1]
    ncls = out_W.shape[1]
    n_chunks = n // _CHUNK
    rem = n - n_chunks * _CHUNK
    rem_base = n_chunks * _CHUNK
    n_workers = 32
    chunks_per_tile = -(-n_chunks // n_workers)

    seg_i = seg.astype(jnp.int32)
    zeros = jnp.zeros((_G, f), jnp.float32)
    partial = _sc_segment_sum(n, f, n_chunks, chunks_per_tile,
                              rem_base, rem)(x, seg_i, zeros)

    outw_p = jnp.zeros((pre, _LANE), jnp.float32).at[:, :ncls].set(out_W)
    outb_p = jnp.zeros((1, _LANE), jnp.float32).at[0, :ncls].set(out_b)
    fc1b2 = fc1_b.reshape(1, pre)

    out_padded = pl.pallas_call(
        _head_kernel,
        in_specs=[
            pl.BlockSpec((2, _G, f), lambda: (0, 0, 0)),
            pl.BlockSpec((f, pre), lambda: (0, 0)),
            pl.BlockSpec((1, pre), lambda: (0, 0)),
            pl.BlockSpec((pre, _LANE), lambda: (0, 0)),
            pl.BlockSpec((1, _LANE), lambda: (0, 0)),
        ],
        out_specs=pl.BlockSpec((_G, _LANE), lambda: (0, 0)),
        out_shape=jax.ShapeDtypeStruct((_G, _LANE), jnp.float32),
    )(partial, fc1_W, fc1b2, outw_p, outb_p)
    return out_padded[:, :ncls]


# speedup vs baseline: 1.1155x; 1.1155x over previous
import functools

import jax
import jax.numpy as jnp
from jax import lax
from jax.experimental import pallas as pl
from jax.experimental.pallas import tpu as pltpu
from jax.experimental.pallas import tpu_sc as plsc

_G = 16
_LANE = 128
_CHUNK = 128


def _sc_segment_sum(n, f, n_chunks, chunks_per_tile, rem_base, rem):
    mesh = plsc.VectorSubcoreMesh(core_axis_name="c", subcore_axis_name="s")
    nc = mesh.num_cores

    @functools.partial(
        pl.kernel,
        out_type=jax.ShapeDtypeStruct((nc, _G, f), jnp.float32),
        mesh=mesh,
        scratch_types=[
            [pltpu.VMEM((_CHUNK, f), jnp.float32)] * chunks_per_tile,
            [pltpu.VMEM((_CHUNK,), jnp.int32)] * chunks_per_tile,
            pltpu.VMEM((rem, f), jnp.float32),
            pltpu.VMEM((rem,), jnp.int32),
            pltpu.VMEM_SHARED((_G, f), jnp.float32),
            [pltpu.SemaphoreType.DMA] * (2 * chunks_per_tile),
            pltpu.SemaphoreType.DMA,
        ],
    )
    def seg_sum(x_hbm, seg_hbm, zeros_hbm, out_hbm,
                chunk_v, idx_v, rem_v, ridx_v, acc_sh, load_sems, scat_sem):
        c_id = lax.axis_index("c")
        s_id = lax.axis_index("s")
        wid = s_id * nc + c_id
        n_workers = nc * mesh.num_subcores

        @pl.when(s_id == 0)
        def _init():
            pltpu.sync_copy(zeros_hbm, acc_sh)

        for k in range(chunks_per_tile):
            cid = wid + k * n_workers

            @pl.when(cid < n_chunks)
            def _fire():
                base = cid * _CHUNK
                pltpu.async_copy(x_hbm.at[pl.ds(base, _CHUNK)],
                                 chunk_v[k], load_sems[2 * k])
                pltpu.async_copy(seg_hbm.at[pl.ds(base, _CHUNK)],
                                 idx_v[k], load_sems[2 * k + 1])

        plsc.subcore_barrier()

        for k in range(chunks_per_tile):
            cid = wid + k * n_workers

            @pl.when(cid < n_chunks)
            def _scatter():
                base = cid * _CHUNK
                pltpu.make_async_copy(x_hbm.at[pl.ds(base, _CHUNK)],
                                      chunk_v[k], load_sems[2 * k]).wait()
                pltpu.make_async_copy(seg_hbm.at[pl.ds(base, _CHUNK)],
                                      idx_v[k], load_sems[2 * k + 1]).wait()
                pltpu.async_copy(chunk_v[k], acc_sh.at[idx_v[k]], scat_sem,
                                 add=True)

        @pl.when(wid == 0)
        def _remainder():
            pltpu.sync_copy(x_hbm.at[pl.ds(rem_base, rem)], rem_v)
            pltpu.sync_copy(seg_hbm.at[pl.ds(rem_base, rem)], ridx_v)
            pltpu.sync_copy(rem_v, acc_sh.at[ridx_v], add=True)

        for k in range(chunks_per_tile):
            cid = wid + k * n_workers

            @pl.when(cid < n_chunks)
            def _drain():
                pltpu.make_async_copy(chunk_v[k], acc_sh.at[idx_v[k]],
                                      scat_sem).wait()

        plsc.subcore_barrier()

        @pl.when(s_id == 0)
        def _flush():
            pltpu.sync_copy(acc_sh, out_hbm.at[c_id])

    return seg_sum


def _head_kernel(part_ref, fc1w_ref, fc1b_ref, outw_ref, outb_ref, o_ref):
    pooled = part_ref[0] + part_ref[1]
    h = jnp.dot(pooled, fc1w_ref[...],
                preferred_element_type=jnp.float32) + fc1b_ref[0, :]
    logits = jnp.dot(h, outw_ref[...],
                     preferred_element_type=jnp.float32) + outb_ref[0, :]
    o_ref[...] = jax.nn.sigmoid(logits)


def kernel(x, edge_index, seg, kernel0, a_self0, a_neigh0, bias0,
           kernel1, a_self1, a_neigh1, bias1, fc1_W, fc1_b, out_W, out_b):
    n, f = x.shape
    pre = fc1_W.shape[1]
    ncls = out_W.shape[1]
    n_chunks = n // _CHUNK
    rem = n - n_chunks * _CHUNK
    rem_base = n_chunks * _CHUNK
    n_workers = 32
    chunks_per_tile = -(-n_chunks // n_workers)

    seg_i = seg.astype(jnp.int32)
    zeros = jnp.zeros((_G, f), jnp.float32)
    partial = _sc_segment_sum(n, f, n_chunks, chunks_per_tile,
                              rem_base, rem)(x, seg_i, zeros)

    outw_p = jnp.zeros((pre, _LANE), jnp.float32).at[:, :ncls].set(out_W)
    outb_p = jnp.zeros((1, _LANE), jnp.float32).at[0, :ncls].set(out_b)
    fc1b2 = fc1_b.reshape(1, pre)

    out_padded = pl.pallas_call(
        _head_kernel,
        in_specs=[
            pl.BlockSpec((2, _G, f), lambda: (0, 0, 0)),
            pl.BlockSpec((f, pre), lambda: (0, 0)),
            pl.BlockSpec((1, pre), lambda: (0, 0)),
            pl.BlockSpec((pre, _LANE), lambda: (0, 0)),
            pl.BlockSpec((1, _LANE), lambda: (0, 0)),
        ],
        out_specs=pl.BlockSpec((_G, _LANE), lambda: (0, 0)),
        out_shape=jax.ShapeDtypeStruct((_G, _LANE), jnp.float32),
    )(partial, fc1_W, fc1b2, outw_p, outb_p)
    return out_padded[:, :ncls]


# speedup vs baseline: 1.1415x; 1.0233x over previous
import functools

import jax
import jax.numpy as jnp
from jax import lax
from jax.experimental import pallas as pl
from jax.experimental.pallas import tpu as pltpu
from jax.experimental.pallas import tpu_sc as plsc

_G = 16
_LANE = 128
_CHUNK = 104


def _sc_segment_sum(n, f, n_chunks, chunks_per_tile, rem_base, rem):
    mesh = plsc.VectorSubcoreMesh(core_axis_name="c", subcore_axis_name="s")
    nc = mesh.num_cores

    @functools.partial(
        pl.kernel,
        out_type=jax.ShapeDtypeStruct((nc, _G, f), jnp.float32),
        mesh=mesh,
        scratch_types=[
            [pltpu.VMEM((_CHUNK, f), jnp.float32)] * chunks_per_tile,
            [pltpu.VMEM((_CHUNK,), jnp.int32)] * chunks_per_tile,
            pltpu.VMEM((rem, f), jnp.float32),
            pltpu.VMEM((rem,), jnp.int32),
            pltpu.VMEM_SHARED((_G, f), jnp.float32),
            [pltpu.SemaphoreType.DMA] * (2 * chunks_per_tile),
            pltpu.SemaphoreType.DMA,
        ],
    )
    def seg_sum(x_hbm, seg_hbm, zeros_hbm, out_hbm,
                chunk_v, idx_v, rem_v, ridx_v, acc_sh, load_sems, scat_sem):
        c_id = lax.axis_index("c")
        s_id = lax.axis_index("s")
        wid = s_id * nc + c_id
        n_workers = nc * mesh.num_subcores

        @pl.when(s_id == 0)
        def _init():
            pltpu.sync_copy(zeros_hbm, acc_sh)

        for k in range(chunks_per_tile):
            cid = wid + k * n_workers

            @pl.when(cid < n_chunks)
            def _fire():
                base = cid * _CHUNK
                pltpu.async_copy(x_hbm.at[pl.ds(base, _CHUNK)],
                                 chunk_v[k], load_sems[2 * k])
                pltpu.async_copy(seg_hbm.at[pl.ds(base, _CHUNK)],
                                 idx_v[k], load_sems[2 * k + 1])

        plsc.subcore_barrier()

        for k in range(chunks_per_tile):
            cid = wid + k * n_workers

            @pl.when(cid < n_chunks)
            def _scatter():
                base = cid * _CHUNK
                pltpu.make_async_copy(x_hbm.at[pl.ds(base, _CHUNK)],
                                      chunk_v[k], load_sems[2 * k]).wait()
                pltpu.make_async_copy(seg_hbm.at[pl.ds(base, _CHUNK)],
                                      idx_v[k], load_sems[2 * k + 1]).wait()
                pltpu.async_copy(chunk_v[k], acc_sh.at[idx_v[k]], scat_sem,
                                 add=True)

        @pl.when(wid == 0)
        def _remainder():
            pltpu.sync_copy(x_hbm.at[pl.ds(rem_base, rem)], rem_v)
            pltpu.sync_copy(seg_hbm.at[pl.ds(rem_base, rem)], ridx_v)
            pltpu.sync_copy(rem_v, acc_sh.at[ridx_v], add=True)

        for k in range(chunks_per_tile):
            cid = wid + k * n_workers

            @pl.when(cid < n_chunks)
            def _drain():
                pltpu.make_async_copy(chunk_v[k], acc_sh.at[idx_v[k]],
                                      scat_sem).wait()

        plsc.subcore_barrier()

        @pl.when(s_id == 0)
        def _flush():
            pltpu.sync_copy(acc_sh, out_hbm.at[c_id])

    return seg_sum


def _head_kernel(part_ref, fc1w_ref, fc1b_ref, outw_ref, outb_ref, o_ref):
    pooled = part_ref[0] + part_ref[1]
    h = jnp.dot(pooled, fc1w_ref[...],
                preferred_element_type=jnp.float32) + fc1b_ref[0, :]
    logits = jnp.dot(h, outw_ref[...],
                     preferred_element_type=jnp.float32) + outb_ref[0, :]
    o_ref[...] = jax.nn.sigmoid(logits)


def kernel(x, edge_index, seg, kernel0, a_self0, a_neigh0, bias0,
           kernel1, a_self1, a_neigh1, bias1, fc1_W, fc1_b, out_W, out_b):
    n, f = x.shape
    pre = fc1_W.shape[1]
    ncls = out_W.shape[1]
    n_chunks = n // _CHUNK
    rem = n - n_chunks * _CHUNK
    rem_base = n_chunks * _CHUNK
    n_workers = 32
    chunks_per_tile = -(-n_chunks // n_workers)

    seg_i = seg.astype(jnp.int32)
    zeros = jnp.zeros((_G, f), jnp.float32)
    partial = _sc_segment_sum(n, f, n_chunks, chunks_per_tile,
                              rem_base, rem)(x, seg_i, zeros)

    outw_p = jnp.zeros((pre, _LANE), jnp.float32).at[:, :ncls].set(out_W)
    outb_p = jnp.zeros((1, _LANE), jnp.float32).at[0, :ncls].set(out_b)
    fc1b2 = fc1_b.reshape(1, pre)

    out_padded = pl.pallas_call(
        _head_kernel,
        in_specs=[
            pl.BlockSpec((2, _G, f), lambda: (0, 0, 0)),
            pl.BlockSpec((f, pre), lambda: (0, 0)),
            pl.BlockSpec((1, pre), lambda: (0, 0)),
            pl.BlockSpec((pre, _LANE), lambda: (0, 0)),
            pl.BlockSpec((1, _LANE), lambda: (0, 0)),
        ],
        out_specs=pl.BlockSpec((_G, _LANE), lambda: (0, 0)),
        out_shape=jax.ShapeDtypeStruct((_G, _LANE), jnp.float32),
    )(partial, fc1_W, fc1b2, outw_p, outb_p)
    return out_padded[:, :ncls]


# speedup vs baseline: 1.1557x; 1.0124x over previous
import functools

import jax
import jax.numpy as jnp
from jax import lax
from jax.experimental import pallas as pl
from jax.experimental.pallas import tpu as pltpu
from jax.experimental.pallas import tpu_sc as plsc

_G = 16
_LANE = 128
_CHUNK = 80


def _sc_segment_sum(n, f, n_chunks, chunks_per_tile):
    mesh = plsc.VectorSubcoreMesh(core_axis_name="c", subcore_axis_name="s")
    nc = mesh.num_cores
    ns = mesh.num_subcores

    @functools.partial(
        pl.kernel,
        out_type=jax.ShapeDtypeStruct((nc, _G, f), jnp.float32),
        mesh=mesh,
        scratch_types=[
            [pltpu.VMEM((_CHUNK, f), jnp.float32)] * chunks_per_tile,
            [pltpu.VMEM((_CHUNK,), jnp.int32)] * chunks_per_tile,
            pltpu.VMEM((_G, f), jnp.float32),
            pltpu.VMEM((_G,), jnp.int32),
            pltpu.VMEM_SHARED((ns, _G, f), jnp.float32),
            [pltpu.SemaphoreType.DMA] * (2 * chunks_per_tile),
            pltpu.SemaphoreType.DMA,
        ],
    )
    def seg_sum(x_hbm, seg_hbm, zeros_hbm, out_hbm,
                chunk_v, idx_v, mrg_v, iota_v, acc_sh, load_sems, scat_sem):
        c_id = lax.axis_index("c")
        s_id = lax.axis_index("s")
        wid = s_id * nc + c_id
        n_workers = nc * ns

        for k in range(chunks_per_tile):
            cid = wid + k * n_workers

            @pl.when(cid < n_chunks)
            def _fire():
                base = cid * _CHUNK
                pltpu.async_copy(x_hbm.at[pl.ds(base, _CHUNK)],
                                 chunk_v[k], load_sems[2 * k])
                pltpu.async_copy(seg_hbm.at[pl.ds(base, _CHUNK)],
                                 idx_v[k], load_sems[2 * k + 1])

        pltpu.sync_copy(zeros_hbm, acc_sh.at[s_id])

        for k in range(chunks_per_tile):
            cid = wid + k * n_workers

            @pl.when(cid < n_chunks)
            def _scatter():
                base = cid * _CHUNK
                pltpu.make_async_copy(x_hbm.at[pl.ds(base, _CHUNK)],
                                      chunk_v[k], load_sems[2 * k]).wait()
                pltpu.make_async_copy(seg_hbm.at[pl.ds(base, _CHUNK)],
                                      idx_v[k], load_sems[2 * k + 1]).wait()
                pltpu.async_copy(chunk_v[k], acc_sh.at[s_id].at[idx_v[k]],
                                 scat_sem, add=True)

        for k in range(chunks_per_tile):
            cid = wid + k * n_workers

            @pl.when(cid < n_chunks)
            def _drain():
                pltpu.make_async_copy(chunk_v[k],
                                      acc_sh.at[s_id].at[idx_v[k]],
                                      scat_sem).wait()

        iota_v[...] = lax.iota(jnp.int32, _G)
        plsc.subcore_barrier()

        @pl.when(s_id > 0)
        def _merge():
            pltpu.sync_copy(acc_sh.at[s_id], mrg_v)
            pltpu.sync_copy(mrg_v, acc_sh.at[0].at[iota_v], add=True)

        plsc.subcore_barrier()

        @pl.when(s_id == 0)
        def _flush():
            pltpu.sync_copy(acc_sh.at[0], out_hbm.at[c_id])

    return seg_sum


def _head_kernel(part_ref, fc1w_ref, fc1b_ref, outw_ref, outb_ref, o_ref):
    pooled = part_ref[0] + part_ref[1]
    h = jnp.dot(pooled, fc1w_ref[...],
                preferred_element_type=jnp.float32) + fc1b_ref[0, :]
    logits = jnp.dot(h, outw_ref[...],
                     preferred_element_type=jnp.float32) + outb_ref[0, :]
    o_ref[...] = jax.nn.sigmoid(logits)


def kernel(x, edge_index, seg, kernel0, a_self0, a_neigh0, bias0,
           kernel1, a_self1, a_neigh1, bias1, fc1_W, fc1_b, out_W, out_b):
    n, f = x.shape
    pre = fc1_W.shape[1]
    ncls = out_W.shape[1]
    n_chunks = n // _CHUNK
    n_workers = 32
    chunks_per_tile = -(-n_chunks // n_workers)

    seg_i = seg.astype(jnp.int32)
    zeros = jnp.zeros((_G, f), jnp.float32)
    partial = _sc_segment_sum(n, f, n_chunks, chunks_per_tile)(x, seg_i, zeros)

    outw_p = jnp.zeros((pre, _LANE), jnp.float32).at[:, :ncls].set(out_W)
    outb_p = jnp.zeros((1, _LANE), jnp.float32).at[0, :ncls].set(out_b)
    fc1b2 = fc1_b.reshape(1, pre)

    out_padded = pl.pallas_call(
        _head_kernel,
        in_specs=[
            pl.BlockSpec((2, _G, f), lambda: (0, 0, 0)),
            pl.BlockSpec((f, pre), lambda: (0, 0)),
            pl.BlockSpec((1, pre), lambda: (0, 0)),
            pl.BlockSpec((pre, _LANE), lambda: (0, 0)),
            pl.BlockSpec((1, _LANE), lambda: (0, 0)),
        ],
        out_specs=pl.BlockSpec((_G, _LANE), lambda: (0, 0)),
        out_shape=jax.ShapeDtypeStruct((_G, _LANE), jnp.float32),
    )(partial, fc1_W, fc1b2, outw_p, outb_p)
    return out_padded[:, :ncls]


# speedup vs baseline: 1.1751x; 1.0168x over previous
import functools

import jax
import jax.numpy as jnp
from jax import lax
from jax.experimental import pallas as pl
from jax.experimental.pallas import tpu as pltpu
from jax.experimental.pallas import tpu_sc as plsc

_G = 16
_LANE = 128
_CHUNK = 80


def _sc_segment_sum(t_off, f, n_chunks, chunks_per_tile):
    mesh = plsc.VectorSubcoreMesh(core_axis_name="c", subcore_axis_name="s")
    nc = mesh.num_cores
    ns = mesh.num_subcores

    @functools.partial(
        pl.kernel,
        out_type=jax.ShapeDtypeStruct((nc, _G, f), jnp.float32),
        mesh=mesh,
        scratch_types=[
            [pltpu.VMEM((_CHUNK, f), jnp.float32)] * chunks_per_tile,
            [pltpu.VMEM((_CHUNK,), jnp.int32)] * chunks_per_tile,
            pltpu.VMEM((_G, f), jnp.float32),
            pltpu.VMEM((_G,), jnp.int32),
            pltpu.VMEM_SHARED((ns, _G, f), jnp.float32),
            [pltpu.SemaphoreType.DMA] * (2 * chunks_per_tile),
            pltpu.SemaphoreType.DMA,
        ],
    )
    def seg_sum(x_hbm, seg_hbm, zeros_hbm, out_hbm,
                chunk_v, idx_v, mrg_v, iota_v, acc_sh, load_sems, scat_sem):
        c_id = lax.axis_index("c")
        s_id = lax.axis_index("s")
        wid = s_id * nc + c_id
        n_workers = nc * ns

        for k in range(chunks_per_tile):
            cid = wid + k * n_workers

            @pl.when(cid < n_chunks)
            def _fire():
                base = t_off + cid * _CHUNK
                pltpu.async_copy(x_hbm.at[pl.ds(base, _CHUNK)],
                                 chunk_v[k], load_sems[2 * k])
                pltpu.async_copy(seg_hbm.at[pl.ds(base, _CHUNK)],
                                 idx_v[k], load_sems[2 * k + 1])

        pltpu.sync_copy(zeros_hbm, acc_sh.at[s_id])

        for k in range(chunks_per_tile):
            cid = wid + k * n_workers

            @pl.when(cid < n_chunks)
            def _scatter():
                base = t_off + cid * _CHUNK
                pltpu.make_async_copy(x_hbm.at[pl.ds(base, _CHUNK)],
                                      chunk_v[k], load_sems[2 * k]).wait()
                pltpu.make_async_copy(seg_hbm.at[pl.ds(base, _CHUNK)],
                                      idx_v[k], load_sems[2 * k + 1]).wait()
                pltpu.async_copy(chunk_v[k], acc_sh.at[s_id].at[idx_v[k]],
                                 scat_sem, add=True)

        for k in range(chunks_per_tile):
            cid = wid + k * n_workers

            @pl.when(cid < n_chunks)
            def _drain():
                pltpu.make_async_copy(chunk_v[k],
                                      acc_sh.at[s_id].at[idx_v[k]],
                                      scat_sem).wait()

        iota_v[...] = lax.iota(jnp.int32, _G)
        plsc.subcore_barrier()

        @pl.when(s_id > 0)
        def _merge():
            pltpu.sync_copy(acc_sh.at[s_id], mrg_v)
            pltpu.sync_copy(mrg_v, acc_sh.at[0].at[iota_v], add=True)

        plsc.subcore_barrier()

        @pl.when(s_id == 0)
        def _flush():
            pltpu.sync_copy(acc_sh.at[0], out_hbm.at[c_id])

    return seg_sum


def _head_kernel(seg_ref, x_ref, part_ref, fc1w_ref, fc1b_ref, outw_ref,
                 outb_ref, o_ref, *, t_rows):
    seg = seg_ref[0, 0, :]
    gids = jax.lax.broadcasted_iota(jnp.int32, (_G, t_rows), 0)
    mask = (seg[None, :] == gids).astype(jnp.float32)
    acc = jnp.dot(mask, x_ref[...], preferred_element_type=jnp.float32)
    pooled = acc + part_ref[0] + part_ref[1]
    h = jnp.dot(pooled, fc1w_ref[...],
                preferred_element_type=jnp.float32) + fc1b_ref[0, :]
    logits = jnp.dot(h, outw_ref[...],
                     preferred_element_type=jnp.float32) + outb_ref[0, :]
    o_ref[...] = jax.nn.sigmoid(logits)


def kernel(x, edge_index, seg, kernel0, a_self0, a_neigh0, bias0,
           kernel1, a_self1, a_neigh1, bias1, fc1_W, fc1_b, out_W, out_b):
    n, f = x.shape
    pre = fc1_W.shape[1]
    ncls = out_W.shape[1]
    t_rows = 4960
    n_chunks = (n - t_rows) // _CHUNK
    n_workers = 32
    chunks_per_tile = -(-n_chunks // n_workers)

    seg_i = seg.astype(jnp.int32)
    zeros = jnp.zeros((_G, f), jnp.float32)
    partial = _sc_segment_sum(t_rows, f, n_chunks,
                              chunks_per_tile)(x, seg_i, zeros)
    seg3 = seg_i[:t_rows].reshape(1, 1, t_rows)

    outw_p = jnp.zeros((pre, _LANE), jnp.float32).at[:, :ncls].set(out_W)
    outb_p = jnp.zeros((1, _LANE), jnp.float32).at[0, :ncls].set(out_b)
    fc1b2 = fc1_b.reshape(1, pre)

    out_padded = pl.pallas_call(
        functools.partial(_head_kernel, t_rows=t_rows),
        grid=(1,),
        in_specs=[
            pl.BlockSpec((1, 1, t_rows), lambda i: (0, 0, 0)),
            pl.BlockSpec((t_rows, f), lambda i: (0, 0)),
            pl.BlockSpec((2, _G, f), lambda i: (0, 0, 0)),
            pl.BlockSpec((f, pre), lambda i: (0, 0)),
            pl.BlockSpec((1, pre), lambda i: (0, 0)),
            pl.BlockSpec((pre, _LANE), lambda i: (0, 0)),
            pl.BlockSpec((1, _LANE), lambda i: (0, 0)),
        ],
        out_specs=pl.BlockSpec((_G, _LANE), lambda i: (0, 0)),
        out_shape=jax.ShapeDtypeStruct((_G, _LANE), jnp.float32),
    )(seg3, x, partial, fc1_W, fc1b2, outw_p, outb_p)
    return out_padded[:, :ncls]
